# baseline (speedup 1.0000x reference)
import functools

import jax
import jax.numpy as jnp
from jax import lax
from jax.experimental import pallas as pl
from jax.experimental.pallas import tpu as pltpu
from jax.experimental.pallas import tpu_sc as plsc

N_NODES = 10000
M_PAD = 10240
N_EDGES = 320000
D_IN = 128
D_HID = 64
D_OUT = 32
D_AUG = 72

NC = 2
NS = 16
NW = NC * NS
CHUNK = 128
E_PAD = 327680
EDGES_PER_TILE = E_PAD // NW
NCHUNK = EDGES_PER_TILE // CHUNK
NBUF = 5
NROUND = NCHUNK // NBUF
ROWS_PER_TILE = M_PAD // NS

ROW_BLK = 2048
GRID_M = M_PAD // ROW_BLK


def _make_agg(depth):
  mesh = plsc.VectorSubcoreMesh(core_axis_name="c", subcore_axis_name="s")

  @functools.partial(
      pl.kernel,
      out_type=jax.ShapeDtypeStruct((NC, M_PAD, depth), jnp.float32),
      mesh=mesh,
      scratch_types=[
          pltpu.VMEM((NCHUNK, CHUNK), jnp.int32),
          pltpu.VMEM((NCHUNK, CHUNK), jnp.int32),
          [pltpu.VMEM((CHUNK, depth), jnp.float32) for _ in range(NBUF)],
          pltpu.VMEM_SHARED((M_PAD, depth), jnp.float32),
          [pltpu.SemaphoreType.DMA for _ in range(NBUF)],
          [pltpu.SemaphoreType.DMA for _ in range(NBUF)],
          pltpu.SemaphoreType.DMA,
      ],
      compiler_params=pltpu.CompilerParams(use_tc_tiling_on_sc=False),
  )
  def agg(table_hbm, src_hbm, dst_hbm, zeros_hbm, out_hbm,
          src_v, dst_v, rows_v, acc_sh, sem_g, sem_s, sem_z):
    core = lax.axis_index("c")
    sub = lax.axis_index("s")
    w = core * NS + sub
    pltpu.sync_copy(src_hbm.at[w], src_v)
    pltpu.sync_copy(dst_hbm.at[w], dst_v)
    r0 = sub * ROWS_PER_TILE
    pltpu.async_copy(zeros_hbm.at[pl.ds(r0, ROWS_PER_TILE)],
                     acc_sh.at[pl.ds(r0, ROWS_PER_TILE)], sem_z).wait()
    plsc.subcore_barrier()

    for b in range(NBUF):
      pltpu.async_copy(table_hbm.at[src_v.at[b]], rows_v[b], sem_g[b])

    @pl.loop(0, NROUND)
    def _(g):
      for b in range(NBUF):
        c = g * NBUF + b
        pltpu.make_async_copy(table_hbm.at[src_v.at[c]],
                              rows_v[b], sem_g[b]).wait()
        pltpu.async_copy(rows_v[b], acc_sh.at[dst_v.at[c]], sem_s[b],
                         add=True)
      @pl.when(g < NROUND - 1)
      def _():
        for b in range(NBUF):
          c = g * NBUF + b
          pltpu.make_async_copy(table_hbm.at[src_v.at[c]],
                                rows_v[b], sem_s[b]).wait()
          pltpu.async_copy(table_hbm.at[src_v.at[c + NBUF]],
                           rows_v[b], sem_g[b])

    for b in range(NBUF):
      pltpu.make_async_copy(table_hbm.at[src_v.at[b]],
                            rows_v[b], sem_s[b]).wait()

    plsc.subcore_barrier()
    pltpu.sync_copy(acc_sh.at[pl.ds(r0, ROWS_PER_TILE)],
                    out_hbm.at[core, pl.ds(r0, ROWS_PER_TILE)])

  return agg


_agg_l1 = _make_agg(D_AUG)
_agg_l2 = _make_agg(D_OUT)


def _mm(a, b):
  return lax.dot_general(a, b, (((1,), (1,)), ((), ())),
                         precision=lax.Precision.HIGHEST,
                         preferred_element_type=jnp.float32)


def _tc1_body(x_ref, w1l_ref, w1r_ref, xla_ref, xr_ref):
  x = x_ref[...]
  xl = _mm(x, w1l_ref[...])
  ones = jnp.ones((ROW_BLK, 1), jnp.float32)
  zeros = jnp.zeros((ROW_BLK, D_AUG - D_HID - 1), jnp.float32)
  xla_ref[...] = jnp.concatenate([xl, ones, zeros], axis=1)
  xr_ref[...] = _mm(x, w1r_ref[...])


def _tc2_body(p1_ref, xr_ref, b1_ref, w2l_ref, w2r_ref, hl_ref, hr_ref):
  s = p1_ref[0] + p1_ref[1]
  agg = s[:, :D_HID]
  deg = jnp.clip(s[:, D_HID:D_HID + 1], 1.0, None)
  h = jnp.maximum(agg / deg + b1_ref[...][None, :] + xr_ref[...], 0.0)
  hl_ref[...] = _mm(h, w2l_ref[...])
  hr_ref[...] = _mm(h, w2r_ref[...])


def _tc3_body(p2_ref, p1_ref, hr_ref, b2_ref, out_ref):
  s2 = p2_ref[0] + p2_ref[1]
  deg = jnp.clip(p1_ref[0, :, D_HID:D_HID + 1] + p1_ref[1, :, D_HID:D_HID + 1],
                 1.0, None)
  out_ref[...] = s2 / deg + b2_ref[...][None, :] + hr_ref[...]


def _tc1(x, W1l, W1r):
  return pl.pallas_call(
      _tc1_body,
      grid=(GRID_M,),
      in_specs=[
          pl.BlockSpec((ROW_BLK, D_IN), lambda i: (i, 0)),
          pl.BlockSpec((D_HID, D_IN), lambda i: (0, 0)),
          pl.BlockSpec((D_HID, D_IN), lambda i: (0, 0)),
      ],
      out_specs=[
          pl.BlockSpec((ROW_BLK, D_AUG), lambda i: (i, 0)),
          pl.BlockSpec((ROW_BLK, D_HID), lambda i: (i, 0)),
      ],
      out_shape=[
          jax.ShapeDtypeStruct((M_PAD, D_AUG), jnp.float32),
          jax.ShapeDtypeStruct((M_PAD, D_HID), jnp.float32),
      ],
  )(x, W1l, W1r)


def _tc2(p1, xr, b1, W2l, W2r):
  return pl.pallas_call(
      _tc2_body,
      grid=(GRID_M,),
      in_specs=[
          pl.BlockSpec((NC, ROW_BLK, D_AUG), lambda i: (0, i, 0)),
          pl.BlockSpec((ROW_BLK, D_HID), lambda i: (i, 0)),
          pl.BlockSpec((D_HID,), lambda i: (0,)),
          pl.BlockSpec((D_OUT, D_HID), lambda i: (0, 0)),
          pl.BlockSpec((D_OUT, D_HID), lambda i: (0, 0)),
      ],
      out_specs=[
          pl.BlockSpec((ROW_BLK, D_OUT), lambda i: (i, 0)),
          pl.BlockSpec((ROW_BLK, D_OUT), lambda i: (i, 0)),
      ],
      out_shape=[
          jax.ShapeDtypeStruct((M_PAD, D_OUT), jnp.float32),
          jax.ShapeDtypeStruct((M_PAD, D_OUT), jnp.float32),
      ],
  )(p1, xr, b1, W2l, W2r)


def _tc3(p2, p1, hr, b2):
  return pl.pallas_call(
      _tc3_body,
      grid=(GRID_M,),
      in_specs=[
          pl.BlockSpec((NC, ROW_BLK, D_OUT), lambda i: (0, i, 0)),
          pl.BlockSpec((NC, ROW_BLK, D_AUG), lambda i: (0, i, 0)),
          pl.BlockSpec((ROW_BLK, D_OUT), lambda i: (i, 0)),
          pl.BlockSpec((D_OUT,), lambda i: (0,)),
      ],
      out_specs=pl.BlockSpec((ROW_BLK, D_OUT), lambda i: (i, 0)),
      out_shape=jax.ShapeDtypeStruct((M_PAD, D_OUT), jnp.float32),
  )(p2, p1, hr, b2)


def kernel(x, edge_index, W1l, b1, W1r, W2l, b2, W2r):
  e0 = edge_index[0].astype(jnp.int32)
  e1 = edge_index[1].astype(jnp.int32)
  npad = E_PAD - N_EDGES
  pad_iota = jnp.arange(npad, dtype=jnp.int32)
  src = jnp.concatenate([e0, pad_iota % N_NODES]).reshape(NW, NCHUNK, CHUNK)
  dst = jnp.concatenate(
      [e1, N_NODES + pad_iota % (M_PAD - N_NODES)]).reshape(NW, NCHUNK, CHUNK)
  xp = jnp.pad(x, ((0, M_PAD - N_NODES), (0, 0)))
  xla_, xr = _tc1(xp, W1l, W1r)
  p1 = _agg_l1(xla_, src, dst, jnp.zeros((M_PAD, D_AUG), jnp.float32))
  hl, hr = _tc2(p1, xr, b1, W2l, W2r)
  p2 = _agg_l2(hl, src, dst, jnp.zeros((M_PAD, D_OUT), jnp.float32))
  return _tc3(p2, p1, hr, b2)[:N_NODES]


# speedup vs baseline: 1.1249x; 1.1249x over previous
import functools

import jax
import jax.numpy as jnp
from jax import lax
from jax.experimental import pallas as pl
from jax.experimental.pallas import tpu as pltpu
from jax.experimental.pallas import tpu_sc as plsc

N_NODES = 10000
M_PAD = 10240
N_EDGES = 320000
D_IN = 128
D_HID = 64
D_OUT = 32
D_DEG = 16

NC = 2
NS = 16
NW = NC * NS
CHUNK = 128
E_PAD = 327680
EDGES_PER_TILE = E_PAD // NW
NCHUNK = EDGES_PER_TILE // CHUNK
NBUF = 5
NROUND = NCHUNK // NBUF
ROWS_PER_TILE = M_PAD // NS

ROW_BLK = 2048
GRID_M = M_PAD // ROW_BLK


def _make_agg(depth, with_deg):
  mesh = plsc.VectorSubcoreMesh(core_axis_name="c", subcore_axis_name="s")

  out_type = [jax.ShapeDtypeStruct((NC, M_PAD, depth), jnp.float32)]
  scratch = [
      pltpu.VMEM((NCHUNK, CHUNK), jnp.int32),
      pltpu.VMEM((NCHUNK, CHUNK), jnp.int32),
      [pltpu.VMEM((CHUNK, depth), jnp.float32) for _ in range(NBUF)],
      pltpu.VMEM_SHARED((M_PAD, depth), jnp.float32),
      [pltpu.SemaphoreType.DMA for _ in range(NBUF)],
      [pltpu.SemaphoreType.DMA for _ in range(NBUF)],
      pltpu.SemaphoreType.DMA,
  ]
  if with_deg:
    out_type.append(jax.ShapeDtypeStruct((NC, M_PAD, D_DEG), jnp.float32))
    scratch += [
        pltpu.VMEM((CHUNK, D_DEG), jnp.float32),
        pltpu.VMEM_SHARED((M_PAD, D_DEG), jnp.float32),
        [pltpu.SemaphoreType.DMA for _ in range(NBUF)],
    ]

  @functools.partial(
      pl.kernel,
      out_type=out_type,
      mesh=mesh,
      scratch_types=scratch,
      compiler_params=pltpu.CompilerParams(use_tc_tiling_on_sc=False),
  )
  def agg(table_hbm, src_hbm, dst_hbm, zeros_hbm, *rest):
    if with_deg:
      (zerosd_hbm, out_hbm, deg_hbm, src_v, dst_v, rows_v, acc_sh,
       sem_g, sem_s, sem_z, ones_v, deg_sh, sem_d) = rest
    else:
      out_hbm, src_v, dst_v, rows_v, acc_sh, sem_g, sem_s, sem_z = rest
    core = lax.axis_index("c")
    sub = lax.axis_index("s")
    w = core * NS + sub
    pltpu.sync_copy(src_hbm.at[w], src_v)
    pltpu.sync_copy(dst_hbm.at[w], dst_v)
    r0 = sub * ROWS_PER_TILE
    pltpu.async_copy(zeros_hbm.at[pl.ds(r0, ROWS_PER_TILE)],
                     acc_sh.at[pl.ds(r0, ROWS_PER_TILE)], sem_z).wait()
    if with_deg:
      pltpu.async_copy(zerosd_hbm.at[pl.ds(r0, ROWS_PER_TILE)],
                       deg_sh.at[pl.ds(r0, ROWS_PER_TILE)], sem_z).wait()
      ones = jnp.ones((D_DEG,), jnp.float32)

      @pl.loop(0, CHUNK)
      def _(i):
        ones_v[i] = ones

    plsc.subcore_barrier()

    for b in range(NBUF):
      pltpu.async_copy(table_hbm.at[src_v.at[b]], rows_v[b], sem_g[b])

    @pl.loop(0, NROUND)
    def _(g):
      for b in range(NBUF):
        c = g * NBUF + b
        pltpu.make_async_copy(table_hbm.at[src_v.at[c]],
                              rows_v[b], sem_g[b]).wait()
        pltpu.async_copy(rows_v[b], acc_sh.at[dst_v.at[c]], sem_s[b],
                         add=True)
        if with_deg:
          pltpu.async_copy(ones_v, deg_sh.at[dst_v.at[c]], sem_d[b],
                           add=True)

        @pl.when(g < NROUND - 1)
        def _():
          pltpu.make_async_copy(table_hbm.at[src_v.at[c]],
                                rows_v[b], sem_s[b]).wait()
          if with_deg:
            pltpu.make_async_copy(zerosd_hbm.at[pl.ds(0, CHUNK)],
                                  ones_v, sem_d[b]).wait()
          pltpu.async_copy(table_hbm.at[src_v.at[c + NBUF]],
                           rows_v[b], sem_g[b])

    for b in range(NBUF):
      pltpu.make_async_copy(table_hbm.at[src_v.at[b]],
                            rows_v[b], sem_s[b]).wait()
      if with_deg:
        pltpu.make_async_copy(zerosd_hbm.at[pl.ds(0, CHUNK)],
                              ones_v, sem_d[b]).wait()

    plsc.subcore_barrier()
    pltpu.sync_copy(acc_sh.at[pl.ds(r0, ROWS_PER_TILE)],
                    out_hbm.at[core, pl.ds(r0, ROWS_PER_TILE)])
    if with_deg:
      pltpu.sync_copy(deg_sh.at[pl.ds(r0, ROWS_PER_TILE)],
                      deg_hbm.at[core, pl.ds(r0, ROWS_PER_TILE)])

  return agg


_agg_l1 = _make_agg(D_HID, True)
_agg_l2 = _make_agg(D_OUT, False)


def _mm(a, b):
  return lax.dot_general(a, b, (((1,), (0,)), ((), ())),
                         precision=lax.Precision.HIGHEST,
                         preferred_element_type=jnp.float32)


def _tc1_body(x2_ref, w1lc_ref, w1rc_ref, xlp_ref, xrp_ref):
  x2 = x2_ref[...]
  xlp_ref[...] = _mm(x2, w1lc_ref[...])
  xrp_ref[...] = _mm(x2, w1rc_ref[...])


def _tc2_body(p1_ref, degp_ref, xrp_ref, b1p_ref, w2lc_ref, w2rc_ref,
              hlp_ref, hrp_ref):
  agg = p1_ref[0] + p1_ref[1]
  h = jnp.maximum(agg / degp_ref[...] + b1p_ref[...][None, :] + xrp_ref[...],
                  0.0)
  hlp_ref[...] = _mm(h, w2lc_ref[...])
  hrp_ref[...] = _mm(h, w2rc_ref[...])


def _tc3_body(p2_ref, degp4_ref, hrp4_ref, b2p4_ref, out_ref):
  s2 = p2_ref[0] + p2_ref[1]
  out_ref[...] = (s2 / degp4_ref[...] + b2p4_ref[...][None, :]
                  + hrp4_ref[...])


def _tc1(x2, W1lc, W1rc):
  return pl.pallas_call(
      _tc1_body,
      grid=(GRID_M,),
      in_specs=[
          pl.BlockSpec((ROW_BLK // 2, 2 * D_IN), lambda i: (i, 0)),
          pl.BlockSpec((2 * D_IN, 128), lambda i: (0, 0)),
          pl.BlockSpec((2 * D_IN, 128), lambda i: (0, 0)),
      ],
      out_specs=[
          pl.BlockSpec((ROW_BLK // 2, 128), lambda i: (i, 0)),
          pl.BlockSpec((ROW_BLK // 2, 128), lambda i: (i, 0)),
      ],
      out_shape=[
          jax.ShapeDtypeStruct((M_PAD // 2, 128), jnp.float32),
          jax.ShapeDtypeStruct((M_PAD // 2, 128), jnp.float32),
      ],
  )(x2, W1lc, W1rc)


def _tc2(p1, degp, xrp, b1p, W2lc, W2rc):
  return pl.pallas_call(
      _tc2_body,
      grid=(GRID_M,),
      in_specs=[
          pl.BlockSpec((NC, ROW_BLK // 2, 128), lambda i: (0, i, 0)),
          pl.BlockSpec((ROW_BLK // 2, 128), lambda i: (i, 0)),
          pl.BlockSpec((ROW_BLK // 2, 128), lambda i: (i, 0)),
          pl.BlockSpec((128,), lambda i: (0,)),
          pl.BlockSpec((128, D_HID), lambda i: (0, 0)),
          pl.BlockSpec((128, D_HID), lambda i: (0, 0)),
      ],
      out_specs=[
          pl.BlockSpec((ROW_BLK // 2, D_HID), lambda i: (i, 0)),
          pl.BlockSpec((ROW_BLK // 2, D_HID), lambda i: (i, 0)),
      ],
      out_shape=[
          jax.ShapeDtypeStruct((M_PAD // 2, D_HID), jnp.float32),
          jax.ShapeDtypeStruct((M_PAD // 2, D_HID), jnp.float32),
      ],
  )(p1, degp, xrp, b1p, W2lc, W2rc)


def _tc3(p2, degp4, hrp4, b2p4):
  return pl.pallas_call(
      _tc3_body,
      grid=(GRID_M,),
      in_specs=[
          pl.BlockSpec((NC, ROW_BLK // 4, 128), lambda i: (0, i, 0)),
          pl.BlockSpec((ROW_BLK // 4, 128), lambda i: (i, 0)),
          pl.BlockSpec((ROW_BLK // 4, 128), lambda i: (i, 0)),
          pl.BlockSpec((128,), lambda i: (0,)),
      ],
      out_specs=pl.BlockSpec((ROW_BLK // 4, 128), lambda i: (i, 0)),
      out_shape=jax.ShapeDtypeStruct((M_PAD // 4, 128), jnp.float32),
  )(p2, degp4, hrp4, b2p4)


def _blockdiag2(w):
  k, n = w.shape
  z = jnp.zeros((k, n), jnp.float32)
  return jnp.concatenate(
      [jnp.concatenate([w, z], axis=1), jnp.concatenate([z, w], axis=1)],
      axis=0)


def kernel(x, edge_index, W1l, b1, W1r, W2l, b2, W2r):
  e0 = edge_index[0].astype(jnp.int32)
  e1 = edge_index[1].astype(jnp.int32)
  npad = E_PAD - N_EDGES
  pad_iota = jnp.arange(npad, dtype=jnp.int32)
  src = jnp.concatenate([e0, pad_iota % N_NODES]).reshape(NW, NCHUNK, CHUNK)
  dst = jnp.concatenate(
      [e1, N_NODES + pad_iota % (M_PAD - N_NODES)]).reshape(NW, NCHUNK, CHUNK)
  x2 = jnp.pad(x, ((0, M_PAD - N_NODES), (0, 0))).reshape(M_PAD // 2, 2 * D_IN)
  W1lc = _blockdiag2(W1l.T)
  W1rc = _blockdiag2(W1r.T)
  W2lc = _blockdiag2(W2l.T)
  W2rc = _blockdiag2(W2r.T)
  b1p = jnp.concatenate([b1, b1])
  b2p4 = jnp.concatenate([b2, b2, b2, b2])

  xlp, xrp = _tc1(x2, W1lc, W1rc)
  xl_tab = xlp.reshape(M_PAD, D_HID)
  p1_out, deg_out = _agg_l1(xl_tab, src, dst,
                            jnp.zeros((M_PAD, D_HID), jnp.float32),
                            jnp.zeros((M_PAD, D_DEG), jnp.float32))
  p1 = p1_out.reshape(NC, M_PAD // 2, 128)
  deg1 = jnp.clip(deg_out[0, :, 0] + deg_out[1, :, 0], 1.0, None)
  degp = jnp.repeat(deg1.reshape(M_PAD // 2, 2), D_HID, axis=1)
  degp4 = jnp.repeat(deg1.reshape(M_PAD // 4, 4), D_OUT, axis=1)

  hlp, hrp = _tc2(p1, degp, xrp, b1p, W2lc, W2rc)
  hl_tab = hlp.reshape(M_PAD, D_OUT)
  hrp4 = hrp.reshape(M_PAD // 4, 128)
  p2_out = _agg_l2(hl_tab, src, dst, jnp.zeros((M_PAD, D_OUT), jnp.float32))
  p2 = p2_out[0].reshape(NC, M_PAD // 4, 128)
  outp = _tc3(p2, degp4, hrp4, b2p4)
  return outp.reshape(M_PAD, D_OUT)[:N_NODES]


# speedup vs baseline: 1.2161x; 1.0810x over previous
import functools

import jax
import jax.numpy as jnp
from jax import lax
from jax.experimental import pallas as pl
from jax.experimental.pallas import tpu as pltpu
from jax.experimental.pallas import tpu_sc as plsc

N_NODES = 10000
M_PAD = 10240
N_EDGES = 320000
D_IN = 128
D_HID = 64
D_OUT = 32
D_DEG = 16

NC = 2
NS = 16
NW = NC * NS
CHUNK = 128
E_PAD = 327680
EDGES_PER_TILE = E_PAD // NW
NCHUNK = EDGES_PER_TILE // CHUNK
NBUF = 5
NROUND = NCHUNK // NBUF
ROWS_PER_TILE = M_PAD // NS

ROW_BLK = 2048
GRID_M = M_PAD // ROW_BLK


def _make_agg(depth, with_deg):
  mesh = plsc.VectorSubcoreMesh(core_axis_name="c", subcore_axis_name="s")

  out_type = [jax.ShapeDtypeStruct((NC, M_PAD, depth), jnp.float32)]
  scratch = [
      pltpu.VMEM((NCHUNK, CHUNK), jnp.int32),
      pltpu.VMEM((NCHUNK, CHUNK), jnp.int32),
      [pltpu.VMEM((CHUNK, depth), jnp.float32) for _ in range(NBUF)],
      pltpu.VMEM_SHARED((M_PAD, depth), jnp.float32),
      [pltpu.SemaphoreType.DMA for _ in range(NBUF)],
      [pltpu.SemaphoreType.DMA for _ in range(NBUF)],
      pltpu.SemaphoreType.DMA,
  ]
  if with_deg:
    out_type.append(jax.ShapeDtypeStruct((NC, NS, M_PAD), jnp.float32))
    scratch.append(pltpu.VMEM((M_PAD,), jnp.float32))

  @functools.partial(
      pl.kernel,
      out_type=out_type,
      mesh=mesh,
      scratch_types=scratch,
      compiler_params=pltpu.CompilerParams(use_tc_tiling_on_sc=False,
                                           needs_layout_passes=False),
  )
  def agg(table_hbm, src_hbm, dst_hbm, zeros_hbm, *rest):
    if with_deg:
      (zeros1_hbm, out_hbm, deg_hbm, src_v, dst_v, rows_v, acc_sh,
       sem_g, sem_s, sem_z, hist_v) = rest
    else:
      out_hbm, src_v, dst_v, rows_v, acc_sh, sem_g, sem_s, sem_z = rest
    core = lax.axis_index("c")
    sub = lax.axis_index("s")
    w = core * NS + sub
    pltpu.sync_copy(src_hbm.at[w], src_v)
    pltpu.sync_copy(dst_hbm.at[w], dst_v)
    r0 = sub * ROWS_PER_TILE
    pltpu.async_copy(zeros_hbm.at[pl.ds(r0, ROWS_PER_TILE)],
                     acc_sh.at[pl.ds(r0, ROWS_PER_TILE)], sem_z).wait()
    if with_deg:
      pltpu.async_copy(zeros1_hbm, hist_v, sem_z).wait()
      ones16 = jnp.ones((16,), jnp.float32)

    plsc.subcore_barrier()

    for b in range(NBUF):
      pltpu.async_copy(table_hbm.at[src_v.at[b]], rows_v[b], sem_g[b])

    @pl.loop(0, NROUND)
    def _(g):
      for b in range(NBUF):
        c = g * NBUF + b
        pltpu.make_async_copy(table_hbm.at[src_v.at[c]],
                              rows_v[b], sem_g[b]).wait()
        pltpu.async_copy(rows_v[b], acc_sh.at[dst_v.at[c]], sem_s[b],
                         add=True)
        if with_deg:
          for k in range(CHUNK // 16):
            idx16 = dst_v[c, pl.ds(k * 16, 16)]
            plsc.addupdate_scatter(hist_v, [idx16], ones16)

        @pl.when(g < NROUND - 1)
        def _():
          pltpu.make_async_copy(table_hbm.at[src_v.at[c]],
                                rows_v[b], sem_s[b]).wait()
          pltpu.async_copy(table_hbm.at[src_v.at[c + NBUF]],
                           rows_v[b], sem_g[b])

    for b in range(NBUF):
      pltpu.make_async_copy(table_hbm.at[src_v.at[b]],
                            rows_v[b], sem_s[b]).wait()

    plsc.subcore_barrier()
    pltpu.sync_copy(acc_sh.at[pl.ds(r0, ROWS_PER_TILE)],
                    out_hbm.at[core, pl.ds(r0, ROWS_PER_TILE)])
    if with_deg:
      pltpu.sync_copy(hist_v, deg_hbm.at[core, sub])

  return agg


_agg_l1 = _make_agg(D_HID, True)
_agg_l2 = _make_agg(D_OUT, False)


def _mm(a, b):
  return lax.dot_general(a, b, (((1,), (0,)), ((), ())),
                         precision=lax.Precision.HIGHEST,
                         preferred_element_type=jnp.float32)


def _tc1_body(x2_ref, w1lc_ref, w1rc_ref, xlp_ref, xrp_ref):
  x2 = x2_ref[...]
  xlp_ref[...] = _mm(x2, w1lc_ref[...])
  xrp_ref[...] = _mm(x2, w1rc_ref[...])


def _tc2_body(p1_ref, degp_ref, xrp_ref, b1p_ref, w2lc_ref, w2rc_ref,
              hlp_ref, hrp_ref):
  agg = p1_ref[0] + p1_ref[1]
  h = jnp.maximum(agg / degp_ref[...] + b1p_ref[...][None, :] + xrp_ref[...],
                  0.0)
  hlp_ref[...] = _mm(h, w2lc_ref[...])
  hrp_ref[...] = _mm(h, w2rc_ref[...])


def _tc3_body(p2_ref, degp4_ref, hrp4_ref, b2p4_ref, out_ref):
  s2 = p2_ref[0] + p2_ref[1]
  out_ref[...] = (s2 / degp4_ref[...] + b2p4_ref[...][None, :]
                  + hrp4_ref[...])


def _tc1(x2, W1lc, W1rc):
  return pl.pallas_call(
      _tc1_body,
      grid=(GRID_M,),
      in_specs=[
          pl.BlockSpec((ROW_BLK // 2, 2 * D_IN), lambda i: (i, 0)),
          pl.BlockSpec((2 * D_IN, 128), lambda i: (0, 0)),
          pl.BlockSpec((2 * D_IN, 128), lambda i: (0, 0)),
      ],
      out_specs=[
          pl.BlockSpec((ROW_BLK // 2, 128), lambda i: (i, 0)),
          pl.BlockSpec((ROW_BLK // 2, 128), lambda i: (i, 0)),
      ],
      out_shape=[
          jax.ShapeDtypeStruct((M_PAD // 2, 128), jnp.float32),
          jax.ShapeDtypeStruct((M_PAD // 2, 128), jnp.float32),
      ],
  )(x2, W1lc, W1rc)


def _tc2(p1, degp, xrp, b1p, W2lc, W2rc):
  return pl.pallas_call(
      _tc2_body,
      grid=(GRID_M,),
      in_specs=[
          pl.BlockSpec((NC, ROW_BLK // 2, 128), lambda i: (0, i, 0)),
          pl.BlockSpec((ROW_BLK // 2, 128), lambda i: (i, 0)),
          pl.BlockSpec((ROW_BLK // 2, 128), lambda i: (i, 0)),
          pl.BlockSpec((128,), lambda i: (0,)),
          pl.BlockSpec((128, D_HID), lambda i: (0, 0)),
          pl.BlockSpec((128, D_HID), lambda i: (0, 0)),
      ],
      out_specs=[
          pl.BlockSpec((ROW_BLK // 2, D_HID), lambda i: (i, 0)),
          pl.BlockSpec((ROW_BLK // 2, D_HID), lambda i: (i, 0)),
      ],
      out_shape=[
          jax.ShapeDtypeStruct((M_PAD // 2, D_HID), jnp.float32),
          jax.ShapeDtypeStruct((M_PAD // 2, D_HID), jnp.float32),
      ],
  )(p1, degp, xrp, b1p, W2lc, W2rc)


def _tc3(p2, degp4, hrp4, b2p4):
  return pl.pallas_call(
      _tc3_body,
      grid=(GRID_M,),
      in_specs=[
          pl.BlockSpec((NC, ROW_BLK // 4, 128), lambda i: (0, i, 0)),
          pl.BlockSpec((ROW_BLK // 4, 128), lambda i: (i, 0)),
          pl.BlockSpec((ROW_BLK // 4, 128), lambda i: (i, 0)),
          pl.BlockSpec((128,), lambda i: (0,)),
      ],
      out_specs=pl.BlockSpec((ROW_BLK // 4, 128), lambda i: (i, 0)),
      out_shape=jax.ShapeDtypeStruct((M_PAD // 4, 128), jnp.float32),
  )(p2, degp4, hrp4, b2p4)


def _blockdiag2(w):
  k, n = w.shape
  z = jnp.zeros((k, n), jnp.float32)
  return jnp.concatenate(
      [jnp.concatenate([w, z], axis=1), jnp.concatenate([z, w], axis=1)],
      axis=0)


def kernel(x, edge_index, W1l, b1, W1r, W2l, b2, W2r):
  e0 = edge_index[0].astype(jnp.int32)
  e1 = edge_index[1].astype(jnp.int32)
  npad = E_PAD - N_EDGES
  pad_iota = jnp.arange(npad, dtype=jnp.int32)
  src = jnp.concatenate([e0, pad_iota % N_NODES]).reshape(NW, NCHUNK, CHUNK)
  dst = jnp.concatenate(
      [e1, N_NODES + pad_iota % (M_PAD - N_NODES)]).reshape(NW, NCHUNK, CHUNK)
  x2 = jnp.concatenate(
      [x, jnp.zeros((M_PAD - N_NODES, D_IN), jnp.float32)]
  ).reshape(M_PAD // 2, 2 * D_IN)
  W1lc = _blockdiag2(W1l.T)
  W1rc = _blockdiag2(W1r.T)
  W2lc = _blockdiag2(W2l.T)
  W2rc = _blockdiag2(W2r.T)
  b1p = jnp.concatenate([b1, b1])
  b2p4 = jnp.concatenate([b2, b2, b2, b2])

  xlp, xrp = _tc1(x2, W1lc, W1rc)
  xl_tab = xlp.reshape(M_PAD, D_HID)
  p1_out, deg_out = _agg_l1(xl_tab, src, dst,
                            jnp.zeros((M_PAD, D_HID), jnp.float32),
                            jnp.zeros((M_PAD,), jnp.float32))
  p1 = p1_out.reshape(NC, M_PAD // 2, 128)
  deg1 = jnp.clip(deg_out.sum(axis=(0, 1)), 1.0, None)
  degp = jnp.repeat(deg1.reshape(M_PAD // 2, 2), D_HID, axis=1)
  degp4 = jnp.repeat(deg1.reshape(M_PAD // 4, 4), D_OUT, axis=1)

  hlp, hrp = _tc2(p1, degp, xrp, b1p, W2lc, W2rc)
  hl_tab = hlp.reshape(M_PAD, D_OUT)
  hrp4 = hrp.reshape(M_PAD // 4, 128)
  p2_out = _agg_l2(hl_tab, src, dst, jnp.zeros((M_PAD, D_OUT), jnp.float32))
  p2 = p2_out[0].reshape(NC, M_PAD // 4, 128)
  outp = _tc3(p2, degp4, hrp4, b2p4)
  return outp.reshape(M_PAD, D_OUT)[:N_NODES]


# speedup vs baseline: 1.2637x; 1.0392x over previous
import functools

import jax
import jax.numpy as jnp
from jax import lax
from jax.experimental import pallas as pl
from jax.experimental.pallas import tpu as pltpu
from jax.experimental.pallas import tpu_sc as plsc

N_NODES = 10000
M_PAD = 10240
N_EDGES = 320000
D_IN = 128
D_HID = 64
D_OUT = 32
D_DEG = 16

NC = 2
NS = 16
NW = NC * NS
CHUNK = 128
E_PAD = 327680
EDGES_PER_TILE = E_PAD // NW
NCHUNK = EDGES_PER_TILE // CHUNK
NBUF = 5
NROUND = NCHUNK // NBUF
ROWS_PER_TILE = M_PAD // NS

ROW_BLK = 2048
GRID_M = M_PAD // ROW_BLK


def _make_agg(depth, with_deg):
  mesh = plsc.VectorSubcoreMesh(core_axis_name="c", subcore_axis_name="s")

  out_type = [jax.ShapeDtypeStruct((NC, M_PAD, depth), jnp.float32)]
  scratch = [
      pltpu.VMEM((NCHUNK, CHUNK), jnp.int32),
      pltpu.VMEM((NCHUNK, CHUNK), jnp.int32),
      [pltpu.VMEM((CHUNK, depth), jnp.float32) for _ in range(NBUF)],
      pltpu.VMEM_SHARED((M_PAD, depth), jnp.float32),
      [pltpu.SemaphoreType.DMA for _ in range(NBUF)],
      [pltpu.SemaphoreType.DMA for _ in range(NBUF)],
      pltpu.SemaphoreType.DMA,
  ]
  if with_deg:
    out_type.append(jax.ShapeDtypeStruct((NC, NS, M_PAD), jnp.float32))
    scratch.append(pltpu.VMEM((M_PAD,), jnp.float32))

  @functools.partial(
      pl.kernel,
      out_type=out_type,
      mesh=mesh,
      scratch_types=scratch,
      compiler_params=pltpu.CompilerParams(use_tc_tiling_on_sc=False,
                                           needs_layout_passes=False),
  )
  def agg(table_hbm, src_hbm, dst_hbm, zeros_hbm, *rest):
    if with_deg:
      (zeros1_hbm, out_hbm, deg_hbm, src_v, dst_v, rows_v, acc_sh,
       sem_g, sem_s, sem_z, hist_v) = rest
    else:
      out_hbm, src_v, dst_v, rows_v, acc_sh, sem_g, sem_s, sem_z = rest
    core = lax.axis_index("c")
    sub = lax.axis_index("s")
    w = core * NS + sub
    pltpu.sync_copy(src_hbm.at[w], src_v)
    pltpu.sync_copy(dst_hbm.at[w], dst_v)
    r0 = sub * ROWS_PER_TILE
    pltpu.async_copy(zeros_hbm.at[pl.ds(r0, ROWS_PER_TILE)],
                     acc_sh.at[pl.ds(r0, ROWS_PER_TILE)], sem_z).wait()
    if with_deg:
      pltpu.async_copy(zeros1_hbm, hist_v, sem_z).wait()
      ones16 = jnp.ones((16,), jnp.float32)

    plsc.subcore_barrier()

    for b in range(NBUF):
      pltpu.async_copy(table_hbm.at[src_v.at[b]], rows_v[b], sem_g[b])

    @pl.loop(0, NROUND)
    def _(g):
      for b in range(NBUF):
        c = g * NBUF + b
        pltpu.make_async_copy(table_hbm.at[src_v.at[c]],
                              rows_v[b], sem_g[b]).wait()
        pltpu.async_copy(rows_v[b], acc_sh.at[dst_v.at[c]], sem_s[b],
                         add=True)
        if with_deg:
          for k in range(CHUNK // 16):
            idx16 = dst_v[c, pl.ds(k * 16, 16)]
            plsc.addupdate_scatter(hist_v, [idx16], ones16)

        @pl.when(g < NROUND - 1)
        def _():
          pltpu.make_async_copy(table_hbm.at[src_v.at[c]],
                                rows_v[b], sem_s[b]).wait()
          pltpu.async_copy(table_hbm.at[src_v.at[c + NBUF]],
                           rows_v[b], sem_g[b])

    for b in range(NBUF):
      pltpu.make_async_copy(table_hbm.at[src_v.at[b]],
                            rows_v[b], sem_s[b]).wait()

    plsc.subcore_barrier()
    pltpu.sync_copy(acc_sh.at[pl.ds(r0, ROWS_PER_TILE)],
                    out_hbm.at[core, pl.ds(r0, ROWS_PER_TILE)])
    if with_deg:
      pltpu.sync_copy(hist_v, deg_hbm.at[core, sub])

  return agg


_agg_l1 = _make_agg(D_HID, True)
_agg_l2 = _make_agg(D_OUT, False)


def _mm(a, b):
  return lax.dot_general(a, b, (((1,), (0,)), ((), ())),
                         precision=lax.Precision.DEFAULT,
                         preferred_element_type=jnp.float32)


def _tc1_body(x2_ref, w1lc_ref, w1rc_ref, xlp_ref, xrp_ref):
  x2 = x2_ref[...]
  xlp_ref[...] = _mm(x2, w1lc_ref[...])
  xrp_ref[...] = _mm(x2, w1rc_ref[...])


def _tc2_body(p1_ref, degp_ref, xrp_ref, b1p_ref, w2lc_ref, w2rc_ref,
              hlp_ref, hrp_ref):
  agg = p1_ref[0] + p1_ref[1]
  h = jnp.maximum(agg / degp_ref[...] + b1p_ref[...][None, :] + xrp_ref[...],
                  0.0)
  hlp_ref[...] = _mm(h, w2lc_ref[...])
  hrp_ref[...] = _mm(h, w2rc_ref[...])


def _tc3_body(p2_ref, degp4_ref, hrp4_ref, b2p4_ref, out_ref):
  s2 = p2_ref[0] + p2_ref[1]
  out_ref[...] = (s2 / degp4_ref[...] + b2p4_ref[...][None, :]
                  + hrp4_ref[...])


def _tc1(x2, W1lc, W1rc):
  return pl.pallas_call(
      _tc1_body,
      grid=(GRID_M,),
      in_specs=[
          pl.BlockSpec((ROW_BLK // 2, 2 * D_IN), lambda i: (i, 0)),
          pl.BlockSpec((2 * D_IN, 128), lambda i: (0, 0)),
          pl.BlockSpec((2 * D_IN, 128), lambda i: (0, 0)),
      ],
      out_specs=[
          pl.BlockSpec((ROW_BLK // 2, 128), lambda i: (i, 0)),
          pl.BlockSpec((ROW_BLK // 2, 128), lambda i: (i, 0)),
      ],
      out_shape=[
          jax.ShapeDtypeStruct((M_PAD // 2, 128), jnp.float32),
          jax.ShapeDtypeStruct((M_PAD // 2, 128), jnp.float32),
      ],
  )(x2, W1lc, W1rc)


def _tc2(p1, degp, xrp, b1p, W2lc, W2rc):
  return pl.pallas_call(
      _tc2_body,
      grid=(GRID_M,),
      in_specs=[
          pl.BlockSpec((NC, ROW_BLK // 2, 128), lambda i: (0, i, 0)),
          pl.BlockSpec((ROW_BLK // 2, 128), lambda i: (i, 0)),
          pl.BlockSpec((ROW_BLK // 2, 128), lambda i: (i, 0)),
          pl.BlockSpec((128,), lambda i: (0,)),
          pl.BlockSpec((128, D_HID), lambda i: (0, 0)),
          pl.BlockSpec((128, D_HID), lambda i: (0, 0)),
      ],
      out_specs=[
          pl.BlockSpec((ROW_BLK // 2, D_HID), lambda i: (i, 0)),
          pl.BlockSpec((ROW_BLK // 2, D_HID), lambda i: (i, 0)),
      ],
      out_shape=[
          jax.ShapeDtypeStruct((M_PAD // 2, D_HID), jnp.float32),
          jax.ShapeDtypeStruct((M_PAD // 2, D_HID), jnp.float32),
      ],
  )(p1, degp, xrp, b1p, W2lc, W2rc)


def _tc3(p2, degp4, hrp4, b2p4):
  return pl.pallas_call(
      _tc3_body,
      grid=(GRID_M,),
      in_specs=[
          pl.BlockSpec((NC, ROW_BLK // 4, 128), lambda i: (0, i, 0)),
          pl.BlockSpec((ROW_BLK // 4, 128), lambda i: (i, 0)),
          pl.BlockSpec((ROW_BLK // 4, 128), lambda i: (i, 0)),
          pl.BlockSpec((128,), lambda i: (0,)),
      ],
      out_specs=pl.BlockSpec((ROW_BLK // 4, 128), lambda i: (i, 0)),
      out_shape=jax.ShapeDtypeStruct((M_PAD // 4, 128), jnp.float32),
  )(p2, degp4, hrp4, b2p4)


def _blockdiag2(w):
  k, n = w.shape
  z = jnp.zeros((k, n), jnp.float32)
  return jnp.concatenate(
      [jnp.concatenate([w, z], axis=1), jnp.concatenate([z, w], axis=1)],
      axis=0)


def kernel(x, edge_index, W1l, b1, W1r, W2l, b2, W2r):
  e0 = edge_index[0].astype(jnp.int32)
  e1 = edge_index[1].astype(jnp.int32)
  npad = E_PAD - N_EDGES
  pad_iota = jnp.arange(npad, dtype=jnp.int32)
  src = jnp.concatenate([e0, pad_iota % N_NODES]).reshape(NW, NCHUNK, CHUNK)
  dst = jnp.concatenate(
      [e1, N_NODES + pad_iota % (M_PAD - N_NODES)]).reshape(NW, NCHUNK, CHUNK)
  x2 = jnp.concatenate(
      [x, jnp.zeros((M_PAD - N_NODES, D_IN), jnp.float32)]
  ).reshape(M_PAD // 2, 2 * D_IN)
  W1lc = _blockdiag2(W1l.T)
  W1rc = _blockdiag2(W1r.T)
  W2lc = _blockdiag2(W2l.T)
  W2rc = _blockdiag2(W2r.T)
  b1p = jnp.concatenate([b1, b1])
  b2p4 = jnp.concatenate([b2, b2, b2, b2])

  xlp, xrp = _tc1(x2, W1lc, W1rc)
  xl_tab = xlp.reshape(M_PAD, D_HID)
  p1_out, deg_out = _agg_l1(xl_tab, src, dst,
                            jnp.zeros((M_PAD, D_HID), jnp.float32),
                            jnp.zeros((M_PAD,), jnp.float32))
  p1 = p1_out.reshape(NC, M_PAD // 2, 128)
  deg1 = jnp.clip(deg_out.sum(axis=(0, 1)), 1.0, None)
  degp = jnp.repeat(deg1.reshape(M_PAD // 2, 2), D_HID, axis=1)
  degp4 = jnp.repeat(deg1.reshape(M_PAD // 4, 4), D_OUT, axis=1)

  hlp, hrp = _tc2(p1, degp, xrp, b1p, W2lc, W2rc)
  hl_tab = hlp.reshape(M_PAD, D_OUT)
  hrp4 = hrp.reshape(M_PAD // 4, 128)
  p2_out = _agg_l2(hl_tab, src, dst, jnp.zeros((M_PAD, D_OUT), jnp.float32))
  p2 = p2_out[0].reshape(NC, M_PAD // 4, 128)
  outp = _tc3(p2, degp4, hrp4, b2p4)
  return outp.reshape(M_PAD, D_OUT)[:N_NODES]
